# baseline (speedup 1.0000x reference)
import functools

import jax
import jax.numpy as jnp
from jax.experimental import pallas as pl
from jax.experimental.pallas import tpu as pltpu

ROW_BLK = 128
NSPLIT = 10


def _ce_rows_kernel(*refs):
    x_refs = refs[:NSPLIT]
    t_ref = refs[NSPLIT]
    out_ref = refs[NSPLIT + 1]
    t = t_ref[...]
    xs = [r[...] for r in x_refs]
    chunk = xs[0].shape[1]
    col = jax.lax.broadcasted_iota(jnp.int32, xs[0].shape, 1)
    m = xs[0].dtype.type(-jnp.inf)
    for x in xs:
        m = jnp.maximum(m, jnp.max(x, axis=1, keepdims=True))
    picked = jnp.zeros_like(m)
    s = jnp.zeros_like(m)
    for k, x in enumerate(xs):
        picked += jnp.sum(jnp.where(col + k * chunk == t, x, 0.0),
                          axis=1, keepdims=True)
        s += jnp.sum(jnp.exp(x - m), axis=1, keepdims=True)
    out_ref[...] = jnp.sum(picked - m - jnp.log(s)).reshape(1, 1, 1)


def _mean_kernel(x_ref, out_ref, *, n_rows):
    out_ref[0, 0] = -jnp.sum(x_ref[...]) / n_rows


def kernel(outputs, targets):
    B, V = outputs.shape
    chunk = V // NSPLIT
    t2 = targets.astype(jnp.int32).reshape(B, 1)

    def vocab_spec(k):
        return pl.BlockSpec((ROW_BLK, chunk), lambda i, k=k: (i, k))

    nblk = B // ROW_BLK
    partial = pl.pallas_call(
        _ce_rows_kernel,
        grid=(nblk,),
        in_specs=[vocab_spec(k) for k in range(NSPLIT)]
        + [pl.BlockSpec((ROW_BLK, 1), lambda i: (i, 0))],
        out_specs=pl.BlockSpec((1, 1, 1), lambda i: (i, 0, 0)),
        out_shape=jax.ShapeDtypeStruct((nblk, 1, 1), jnp.float32),
        compiler_params=pltpu.CompilerParams(
            dimension_semantics=("parallel",),
            vmem_limit_bytes=56 * 1024 * 1024,
        ),
    )(*([outputs] * NSPLIT), t2)

    loss = pl.pallas_call(
        functools.partial(_mean_kernel, n_rows=B),
        in_specs=[pl.BlockSpec((nblk, 1, 1), lambda: (0, 0, 0))],
        out_specs=pl.BlockSpec(memory_space=pltpu.SMEM),
        out_shape=jax.ShapeDtypeStruct((1, 1), jnp.float32),
    )(partial)
    return loss[0, 0]


# speedup vs baseline: 1.0156x; 1.0156x over previous
import functools

import jax
import jax.numpy as jnp
from jax.experimental import pallas as pl
from jax.experimental.pallas import tpu as pltpu

ROW_BLK = 128
NCHUNK = 5
DEPTH = 4


def _ce_core_kernel(x_hbm, t_ref, out_ref, buf, acc, m_run, s_run, p_run,
                    sem, *, rows_per_core, vocab):
    chunk = vocab // NCHUNK
    nblk = rows_per_core // ROW_BLK
    nsteps = nblk * NCHUNK
    core = pl.program_id(0)
    row0 = core * rows_per_core

    def start(t):
        rb = t // NCHUNK
        ch = t % NCHUNK
        pltpu.make_async_copy(
            x_hbm.at[pl.ds(row0 + rb * ROW_BLK, ROW_BLK),
                     pl.ds(ch * chunk, chunk)],
            buf.at[t % DEPTH],
            sem.at[t % DEPTH],
        ).start()

    def wait(t):
        rb = t // NCHUNK
        ch = t % NCHUNK
        pltpu.make_async_copy(
            x_hbm.at[pl.ds(row0 + rb * ROW_BLK, ROW_BLK),
                     pl.ds(ch * chunk, chunk)],
            buf.at[t % DEPTH],
            sem.at[t % DEPTH],
        ).wait()

    for t in range(DEPTH - 1):
        start(t)

    acc[...] = jnp.zeros_like(acc)
    col = jax.lax.broadcasted_iota(jnp.int32, (ROW_BLK, chunk), 1)

    def body(t, _):
        rb = t // NCHUNK
        ch = t % NCHUNK

        @pl.when(t + DEPTH - 1 < nsteps)
        def _():
            start(t + DEPTH - 1)

        @pl.when(ch == 0)
        def _():
            m_run[...] = jnp.full_like(m_run, -jnp.inf)
            s_run[...] = jnp.zeros_like(s_run)
            p_run[...] = jnp.zeros_like(p_run)

        wait(t)
        x = buf[t % DEPTH]
        t_blk = t_ref[pl.ds(rb * ROW_BLK, ROW_BLK), :]
        m_old = m_run[...]
        m_new = jnp.maximum(m_old, jnp.max(x, axis=1, keepdims=True))
        s_run[...] = (s_run[...] * jnp.exp(m_old - m_new)
                      + jnp.sum(jnp.exp(x - m_new), axis=1, keepdims=True))
        m_run[...] = m_new
        p_run[...] += jnp.sum(
            jnp.where(col + ch * chunk == t_blk, x, 0.0),
            axis=1, keepdims=True)

        @pl.when(ch == NCHUNK - 1)
        def _():
            acc[...] += p_run[...] - m_run[...] - jnp.log(s_run[...])

        return ()

    jax.lax.fori_loop(0, nsteps, body, ())
    out_ref[...] = jnp.sum(acc[...]).reshape(1, 1, 1)


def _mean_kernel(x_ref, out_ref, *, n_rows):
    out_ref[0, 0] = -jnp.sum(x_ref[...]) / n_rows


def kernel(outputs, targets):
    B, V = outputs.shape
    rows_per_core = B // 2
    t2 = targets.astype(jnp.int32).reshape(B, 1)

    partial = pl.pallas_call(
        functools.partial(_ce_core_kernel, rows_per_core=rows_per_core,
                          vocab=V),
        grid=(2,),
        in_specs=[
            pl.BlockSpec(memory_space=pl.ANY),
            pl.BlockSpec((rows_per_core, 1), lambda c: (c, 0)),
        ],
        out_specs=pl.BlockSpec((1, 1, 1), lambda c: (c, 0, 0)),
        out_shape=jax.ShapeDtypeStruct((2, 1, 1), jnp.float32),
        scratch_shapes=[
            pltpu.VMEM((DEPTH, ROW_BLK, V // NCHUNK), jnp.float32),
            pltpu.VMEM((ROW_BLK, 1), jnp.float32),
            pltpu.VMEM((ROW_BLK, 1), jnp.float32),
            pltpu.VMEM((ROW_BLK, 1), jnp.float32),
            pltpu.VMEM((ROW_BLK, 1), jnp.float32),
            pltpu.SemaphoreType.DMA((DEPTH,)),
        ],
        compiler_params=pltpu.CompilerParams(
            dimension_semantics=("parallel",),
            vmem_limit_bytes=56 * 1024 * 1024,
        ),
    )(outputs, t2)

    loss = pl.pallas_call(
        functools.partial(_mean_kernel, n_rows=B),
        in_specs=[pl.BlockSpec((2, 1, 1), lambda: (0, 0, 0))],
        out_specs=pl.BlockSpec(memory_space=pltpu.SMEM),
        out_shape=jax.ShapeDtypeStruct((1, 1), jnp.float32),
    )(partial)
    return loss[0, 0]
